# baseline (speedup 1.0000x reference)
import jax
import jax.numpy as jnp
from jax.experimental import pallas as pl
from jax.experimental.pallas import tpu as pltpu

_NUM_SLOTS = 8
_RANK = 64
_TOKENS = 4096
_D_IN = 2048
_D_OUT = 4096
_BT = 256


def _lora_block_kernel(slot_smem, x_ref, slots_ref, a_ref, b_ref, o_ref):
    i = pl.program_id(0)
    e_lo = slot_smem[i * _BT]
    e_hi = slot_smem[i * _BT + _BT - 1]
    x = x_ref[...]
    slots = slots_ref[...]

    def body(e, acc):
        inter = jnp.dot(x, a_ref[e], preferred_element_type=jnp.float32)
        mask = (slots == e).astype(jnp.float32)
        inter = (inter * mask).astype(jnp.bfloat16)
        return acc + jnp.dot(inter, b_ref[e], preferred_element_type=jnp.float32)

    acc = jax.lax.fori_loop(
        e_lo, e_hi + 1, body, jnp.zeros((_BT, _D_OUT), jnp.float32)
    )
    o_ref[...] = acc


def kernel(x, lora_a, lora_b, slot_ids):
    slot_ids = slot_ids.astype(jnp.int32)
    slots2d = slot_ids.reshape(_TOKENS, 1)
    x = x.astype(jnp.bfloat16)
    lora_a = lora_a.astype(jnp.bfloat16)
    lora_b = lora_b.astype(jnp.bfloat16)
    grid_spec = pltpu.PrefetchScalarGridSpec(
        num_scalar_prefetch=1,
        grid=(_TOKENS // _BT,),
        in_specs=[
            pl.BlockSpec((_BT, _D_IN), lambda i, s: (i, 0)),
            pl.BlockSpec((_BT, 1), lambda i, s: (i, 0)),
            pl.BlockSpec((_NUM_SLOTS, _D_IN, _RANK), lambda i, s: (0, 0, 0)),
            pl.BlockSpec((_NUM_SLOTS, _RANK, _D_OUT), lambda i, s: (0, 0, 0)),
        ],
        out_specs=pl.BlockSpec((_BT, _D_OUT), lambda i, s: (i, 0)),
    )
    return pl.pallas_call(
        _lora_block_kernel,
        grid_spec=grid_spec,
        out_shape=jax.ShapeDtypeStruct((_TOKENS, _D_OUT), jnp.float32),
    )(slot_ids, x, slots2d, lora_a, lora_b)


# speedup vs baseline: 1.2143x; 1.2143x over previous
import jax
import jax.numpy as jnp
from jax.experimental import pallas as pl
from jax.experimental.pallas import tpu as pltpu

_NUM_SLOTS = 8
_RANK = 64
_TOKENS = 4096
_D_IN = 2048
_D_OUT = 4096
_BT = 256


def _lora_block_kernel(slot_smem, x_ref, slots_ref, a_ref, b_ref, o_ref):
    i = pl.program_id(0)
    e_lo = slot_smem[i * _BT]
    e_hi = slot_smem[i * _BT + _BT - 1]
    x = x_ref[...].astype(jnp.bfloat16)
    slots = slots_ref[...]

    def body(e, acc):
        a_e = a_ref[e].astype(jnp.bfloat16)
        inter = jnp.dot(x, a_e, preferred_element_type=jnp.float32)
        mask = (slots == e).astype(jnp.float32)
        inter = (inter * mask).astype(jnp.bfloat16)
        b_e = b_ref[e].astype(jnp.bfloat16)
        return acc + jnp.dot(inter, b_e, preferred_element_type=jnp.float32)

    acc = jax.lax.fori_loop(
        e_lo, e_hi + 1, body, jnp.zeros((_BT, _D_OUT), jnp.float32)
    )
    o_ref[...] = acc


def kernel(x, lora_a, lora_b, slot_ids):
    slot_ids = slot_ids.astype(jnp.int32)
    slots2d = slot_ids.reshape(_TOKENS, 1)
    grid_spec = pltpu.PrefetchScalarGridSpec(
        num_scalar_prefetch=1,
        grid=(_TOKENS // _BT,),
        in_specs=[
            pl.BlockSpec((_BT, _D_IN), lambda i, s: (i, 0)),
            pl.BlockSpec((_BT, 1), lambda i, s: (i, 0)),
            pl.BlockSpec((_NUM_SLOTS, _D_IN, _RANK), lambda i, s: (0, 0, 0)),
            pl.BlockSpec((_NUM_SLOTS, _RANK, _D_OUT), lambda i, s: (0, 0, 0)),
        ],
        out_specs=pl.BlockSpec((_BT, _D_OUT), lambda i, s: (i, 0)),
    )
    return pl.pallas_call(
        _lora_block_kernel,
        grid_spec=grid_spec,
        out_shape=jax.ShapeDtypeStruct((_TOKENS, _D_OUT), jnp.float32),
    )(slot_ids, x, slots2d, lora_a, lora_b)


# speedup vs baseline: 1.2622x; 1.0395x over previous
import jax
import jax.numpy as jnp
from jax.experimental import pallas as pl
from jax.experimental.pallas import tpu as pltpu

_NUM_SLOTS = 8
_RANK = 64
_TOKENS = 4096
_D_IN = 2048
_D_OUT = 4096
_BT = 256


def _lora_block_kernel(slot_smem, x_ref, slots_ref, a_ref, b_ref, o_ref,
                       a_bf, b_bf):
    i = pl.program_id(0)

    @pl.when(i == 0)
    def _cast_weights():
        a_bf[...] = a_ref[...].astype(jnp.bfloat16)
        b_bf[...] = b_ref[...].astype(jnp.bfloat16)

    e_lo = slot_smem[i * _BT]
    e_hi = slot_smem[i * _BT + _BT - 1]
    x = x_ref[...].astype(jnp.bfloat16)

    @pl.when(e_lo == e_hi)
    def _single_slot():
        inter = jnp.dot(x, a_bf[e_lo], preferred_element_type=jnp.float32)
        o_ref[...] = jnp.dot(inter.astype(jnp.bfloat16), b_bf[e_lo],
                             preferred_element_type=jnp.float32)

    @pl.when(e_lo != e_hi)
    def _boundary():
        slots = slots_ref[...]

        def body(e, acc):
            inter = jnp.dot(x, a_bf[e], preferred_element_type=jnp.float32)
            mask = (slots == e).astype(jnp.float32)
            inter = (inter * mask).astype(jnp.bfloat16)
            return acc + jnp.dot(inter, b_bf[e],
                                 preferred_element_type=jnp.float32)

        o_ref[...] = jax.lax.fori_loop(
            e_lo, e_hi + 1, body, jnp.zeros((_BT, _D_OUT), jnp.float32)
        )


def kernel(x, lora_a, lora_b, slot_ids):
    slot_ids = slot_ids.astype(jnp.int32)
    slots2d = slot_ids.reshape(_TOKENS, 1)
    grid_spec = pltpu.PrefetchScalarGridSpec(
        num_scalar_prefetch=1,
        grid=(_TOKENS // _BT,),
        in_specs=[
            pl.BlockSpec((_BT, _D_IN), lambda i, s: (i, 0)),
            pl.BlockSpec((_BT, 1), lambda i, s: (i, 0)),
            pl.BlockSpec((_NUM_SLOTS, _D_IN, _RANK), lambda i, s: (0, 0, 0)),
            pl.BlockSpec((_NUM_SLOTS, _RANK, _D_OUT), lambda i, s: (0, 0, 0)),
        ],
        out_specs=pl.BlockSpec((_BT, _D_OUT), lambda i, s: (i, 0)),
        scratch_shapes=[
            pltpu.VMEM((_NUM_SLOTS, _D_IN, _RANK), jnp.bfloat16),
            pltpu.VMEM((_NUM_SLOTS, _RANK, _D_OUT), jnp.bfloat16),
        ],
    )
    return pl.pallas_call(
        _lora_block_kernel,
        grid_spec=grid_spec,
        out_shape=jax.ShapeDtypeStruct((_TOKENS, _D_OUT), jnp.float32),
    )(slot_ids, x, slots2d, lora_a, lora_b)
